# Initial kernel scaffold; baseline (speedup 1.0000x reference)
import jax
import jax.numpy as jnp
from jax.experimental import pallas as pl


def kernel(x, edge_index, edge_attr, W1, b1, W2, b2):
    raise NotImplementedError("write your pallas kernel here")


# speedup vs baseline: 4.4104x; 4.4104x over previous
import functools

import jax
import jax.numpy as jnp
from jax import lax
from jax.experimental import pallas as pl
from jax.experimental.pallas import tpu as pltpu
from jax.experimental.pallas import tpu_sc as plsc

N_NODES = 10000
NODE_SIZE = 128
EDGE_SIZE = 16
OUT_CHANNELS = 32
N_EDGES = 320000

NC = 2
NS = 16
NW = NC * NS
E_PER_W = N_EDGES // NW
CHUNK = 80
NCHUNK = E_PER_W // CHUNK
ROWS_PER_TILE = N_NODES // NS
D_SC = 40


def _ab_body(x_ref, wa_ref, wb_ref, a_ref, b_ref):
    xv = x_ref[...]
    a_ref[...] = jnp.dot(xv, wa_ref[...], preferred_element_type=jnp.float32)
    b_ref[...] = jnp.dot(xv, wb_ref[...], preferred_element_type=jnp.float32)


def _compute_ab(x, w1a, w1b):
    return pl.pallas_call(
        _ab_body,
        out_shape=(
            jax.ShapeDtypeStruct((N_NODES, OUT_CHANNELS), jnp.float32),
            jax.ShapeDtypeStruct((N_NODES, OUT_CHANNELS), jnp.float32),
        ),
    )(x, w1a, w1b)


_E_BLK = 8000


def _e_body(ea_ref, we_ref, b1_ref, e_ref):
    e_ref[...] = (
        jnp.dot(ea_ref[...], we_ref[...], preferred_element_type=jnp.float32)
        + b1_ref[...]
    )


def _compute_e(edge_attr, w1e, b1row):
    grid = (N_EDGES // _E_BLK,)
    return pl.pallas_call(
        _e_body,
        grid=grid,
        in_specs=[
            pl.BlockSpec((_E_BLK, EDGE_SIZE), lambda i: (i, 0)),
            pl.BlockSpec((EDGE_SIZE, OUT_CHANNELS), lambda i: (0, 0)),
            pl.BlockSpec((1, OUT_CHANNELS), lambda i: (0, 0)),
        ],
        out_specs=pl.BlockSpec((_E_BLK, OUT_CHANNELS), lambda i: (i, 0)),
        out_shape=jax.ShapeDtypeStruct((N_EDGES, OUT_CHANNELS), jnp.float32),
    )(edge_attr, w1e, b1row)


def _sc_body(a_hbm, b_hbm, e_hbm, idxi_hbm, idxj_hbm, zeros_hbm, hinit_hbm,
             out_hbm, idxi_v, idxj_v, a_buf, b_buf, e_buf, h_buf, stage_v,
             acc, sem_a, sem_b, sem_e):
    c = lax.axis_index("c")
    s = lax.axis_index("s")
    wid = c * NS + s

    pltpu.sync_copy(idxi_hbm.at[wid], idxi_v)
    pltpu.sync_copy(idxj_hbm.at[wid], idxj_v)
    pltpu.sync_copy(hinit_hbm, h_buf)
    pltpu.sync_copy(zeros_hbm, stage_v)
    row0 = s * ROWS_PER_TILE
    e_base = wid * E_PER_W

    if True:
        def chunk_body(g, carry):
            idx_i = idxi_v.at[g]
            idx_j = idxj_v.at[g]
            cp_a = pltpu.async_copy(a_hbm.at[idx_i], a_buf, sem_a)
            cp_b = pltpu.async_copy(b_hbm.at[idx_j], b_buf, sem_b)
            cp_e = pltpu.async_copy(
                e_hbm.at[pl.ds(e_base + g * CHUNK, CHUNK)], e_buf, sem_e)
            cp_a.wait()
            cp_b.wait()
            cp_e.wait()

            def row_body(r, carry2):
                lo = pl.ds(0, 16)
                hi = pl.ds(16, 16)
                h_buf[r, lo] = jnp.maximum(
                    a_buf[r, lo] + b_buf[r, lo] + e_buf[r, lo], 0.0)
                h_buf[r, hi] = jnp.maximum(
                    a_buf[r, hi] + b_buf[r, hi] + e_buf[r, hi], 0.0)
                return carry2

            lax.fori_loop(0, CHUNK, row_body, 0, unroll=4)
            pltpu.sync_copy(h_buf, acc.at[idx_i], add=True)
            return carry

        pltpu.sync_copy(stage_v, acc.at[pl.ds(row0, ROWS_PER_TILE)])
        plsc.subcore_barrier()
        lax.fori_loop(0, NCHUNK, chunk_body, 0)
        plsc.subcore_barrier()
        pltpu.sync_copy(acc.at[pl.ds(row0, ROWS_PER_TILE)], stage_v)
        pltpu.sync_copy(stage_v, out_hbm.at[c, s])


def _sc_scatter(a, b, e, idxi3, idxj3, zeros_tile, hinit):
    mesh = plsc.VectorSubcoreMesh(core_axis_name="c", subcore_axis_name="s")
    kfn = pl.kernel(
        _sc_body,
        out_type=jax.ShapeDtypeStruct((NC, NS, ROWS_PER_TILE, D_SC),
                                      jnp.float32),
        mesh=mesh,
        scratch_types=[
            pltpu.VMEM((NCHUNK, CHUNK), jnp.int32),
            pltpu.VMEM((NCHUNK, CHUNK), jnp.int32),
            pltpu.VMEM((CHUNK, OUT_CHANNELS), jnp.float32),
            pltpu.VMEM((CHUNK, OUT_CHANNELS), jnp.float32),
            pltpu.VMEM((CHUNK, OUT_CHANNELS), jnp.float32),
            pltpu.VMEM((CHUNK, D_SC), jnp.float32),
            pltpu.VMEM((ROWS_PER_TILE, D_SC), jnp.float32),
            pltpu.VMEM_SHARED((N_NODES, D_SC), jnp.float32),
            pltpu.SemaphoreType.DMA,
            pltpu.SemaphoreType.DMA,
            pltpu.SemaphoreType.DMA,
        ],
        compiler_params=pltpu.CompilerParams(use_tc_tiling_on_sc=False),
    )
    return kfn(a, b, e, idxi3, idxj3, zeros_tile, hinit)


def _fin_body(p_ref, w2_ref, b2_ref, o_ref):
    t = p_ref[0] + p_ref[1]
    s = t[:, :OUT_CHANNELS]
    cnt = t[:, OUT_CHANNELS:OUT_CHANNELS + 1]
    o_ref[...] = (
        jnp.dot(s, w2_ref[...], preferred_element_type=jnp.float32)
        + cnt * b2_ref[...]
    )


def _finalize(parts, w2, b2row):
    return pl.pallas_call(
        _fin_body,
        out_shape=jax.ShapeDtypeStruct((N_NODES, OUT_CHANNELS), jnp.float32),
    )(parts, w2, b2row)


def kernel(x, edge_index, edge_attr, W1, b1, W2, b2):
    w1a = W1[:NODE_SIZE]
    w1b = W1[NODE_SIZE:2 * NODE_SIZE]
    w1e = W1[2 * NODE_SIZE:]
    a, b = _compute_ab(x, w1a, w1b)
    e = _compute_e(edge_attr, w1e, b1.reshape(1, OUT_CHANNELS))

    idxi3 = edge_index[0].reshape(NW, NCHUNK, CHUNK)
    idxj3 = edge_index[1].reshape(NW, NCHUNK, CHUNK)
    zeros_tile = jnp.zeros((ROWS_PER_TILE, D_SC), jnp.float32)
    hinit = jnp.zeros((CHUNK, D_SC), jnp.float32).at[:, OUT_CHANNELS].set(1.0)

    parts = _sc_scatter(a, b, e, idxi3, idxj3, zeros_tile, hinit)
    parts = parts.reshape(NC, N_NODES, D_SC)

    return _finalize(parts, W2, b2.reshape(1, OUT_CHANNELS))


# speedup vs baseline: 6.3930x; 1.4495x over previous
import functools

import jax
import jax.numpy as jnp
from jax import lax
from jax.experimental import pallas as pl
from jax.experimental.pallas import tpu as pltpu
from jax.experimental.pallas import tpu_sc as plsc

N_NODES = 10000
NODE_SIZE = 128
EDGE_SIZE = 16
OUT_CHANNELS = 32
N_EDGES = 320000

NC = 2
NS = 16
NW = NC * NS
E_PER_W = N_EDGES // NW
CHUNK = 80
NCHUNK = E_PER_W // CHUNK
ROWS_PER_TILE = N_NODES // NS
D_SC = 40


def _ab_body(x_ref, wa_ref, wb_ref, a_ref, b_ref):
    xv = x_ref[...]
    a_ref[...] = jnp.dot(xv, wa_ref[...], preferred_element_type=jnp.float32)
    b_ref[...] = jnp.dot(xv, wb_ref[...], preferred_element_type=jnp.float32)


def _compute_ab(x, w1a, w1b):
    return pl.pallas_call(
        _ab_body,
        out_shape=(
            jax.ShapeDtypeStruct((N_NODES, OUT_CHANNELS), jnp.float32),
            jax.ShapeDtypeStruct((N_NODES, OUT_CHANNELS), jnp.float32),
        ),
    )(x, w1a, w1b)


_E_ROWS = N_EDGES // 4
_E_BLK = 16000


def _e_body(eat3_ref, we_ref, b1_ref, e_ref):
    parts = []
    for k in range(4):
        parts.append(lax.dot_general(
            eat3_ref[:, k, :], we_ref[...],
            dimension_numbers=(((0,), (0,)), ((), ())),
            preferred_element_type=jnp.float32,
        ))
    e_ref[...] = jnp.concatenate(parts, axis=1) + b1_ref[...]


def _compute_e(edge_attr_t3, w1e, b1row128):
    grid = (_E_ROWS // _E_BLK,)
    return pl.pallas_call(
        _e_body,
        grid=grid,
        in_specs=[
            pl.BlockSpec((EDGE_SIZE, 4, _E_BLK), lambda i: (0, 0, i)),
            pl.BlockSpec((EDGE_SIZE, OUT_CHANNELS), lambda i: (0, 0)),
            pl.BlockSpec((1, 128), lambda i: (0, 0)),
        ],
        out_specs=pl.BlockSpec((_E_BLK, 128), lambda i: (i, 0)),
        out_shape=jax.ShapeDtypeStruct((_E_ROWS, 128), jnp.float32),
    )(edge_attr_t3, w1e, b1row128)


def _sc_body(a_hbm, b_hbm, e_hbm, idxi_hbm, idxj_hbm, zeros_hbm, hinit_hbm,
             out_hbm, idxi_v, idxj_v, a_buf, b_buf, e_buf, h_buf, stage_v,
             acc, sem_a, sem_b, sem_e):
    c = lax.axis_index("c")
    s = lax.axis_index("s")
    wid = c * NS + s

    pltpu.sync_copy(idxi_hbm.at[wid], idxi_v)
    pltpu.sync_copy(idxj_hbm.at[wid], idxj_v)
    pltpu.sync_copy(hinit_hbm, h_buf)
    pltpu.sync_copy(zeros_hbm, stage_v)
    row0 = s * ROWS_PER_TILE
    e_col0 = (wid // 8) * OUT_CHANNELS
    e_row0 = (wid % 8) * E_PER_W

    if True:
        def chunk_body(g, carry):
            idx_i = idxi_v.at[g]
            idx_j = idxj_v.at[g]
            cp_a = pltpu.async_copy(a_hbm.at[idx_i], a_buf, sem_a)
            cp_b = pltpu.async_copy(b_hbm.at[idx_j], b_buf, sem_b)
            cp_e = pltpu.async_copy(
                e_hbm.at[pl.ds(e_row0 + g * CHUNK, CHUNK),
                         pl.ds(e_col0, OUT_CHANNELS)],
                e_buf, sem_e)
            cp_a.wait()
            cp_b.wait()
            cp_e.wait()

            def row_body(r, carry2):
                for hh in (0, 16):
                    sl = pl.ds(hh, 16)
                    h_buf[r, sl] = jnp.maximum(
                        a_buf[r, sl] + b_buf[r, sl] + e_buf[r, sl], 0.0)
                return carry2

            lax.fori_loop(0, CHUNK, row_body, 0, unroll=4)
            pltpu.sync_copy(h_buf, acc.at[idx_i], add=True)
            return carry

        pltpu.sync_copy(stage_v, acc.at[pl.ds(row0, ROWS_PER_TILE)])
        plsc.subcore_barrier()
        lax.fori_loop(0, NCHUNK, chunk_body, 0)
        plsc.subcore_barrier()
        pltpu.sync_copy(acc.at[pl.ds(row0, ROWS_PER_TILE)], stage_v)
        pltpu.sync_copy(stage_v, out_hbm.at[c, s])


def _sc_scatter(a, b, e, idxi3, idxj3, zeros_tile, hinit):
    mesh = plsc.VectorSubcoreMesh(core_axis_name="c", subcore_axis_name="s")
    kfn = pl.kernel(
        _sc_body,
        out_type=jax.ShapeDtypeStruct((NC, NS, ROWS_PER_TILE, D_SC),
                                      jnp.float32),
        mesh=mesh,
        scratch_types=[
            pltpu.VMEM((NCHUNK, CHUNK), jnp.int32),
            pltpu.VMEM((NCHUNK, CHUNK), jnp.int32),
            pltpu.VMEM((CHUNK, OUT_CHANNELS), jnp.float32),
            pltpu.VMEM((CHUNK, OUT_CHANNELS), jnp.float32),
            pltpu.VMEM((CHUNK, OUT_CHANNELS), jnp.float32),
            pltpu.VMEM((CHUNK, D_SC), jnp.float32),
            pltpu.VMEM((ROWS_PER_TILE, D_SC), jnp.float32),
            pltpu.VMEM_SHARED((N_NODES, D_SC), jnp.float32),
            pltpu.SemaphoreType.DMA,
            pltpu.SemaphoreType.DMA,
            pltpu.SemaphoreType.DMA,
        ],
        compiler_params=pltpu.CompilerParams(use_tc_tiling_on_sc=False),
    )
    return kfn(a, b, e, idxi3, idxj3, zeros_tile, hinit)


def _fin_body(p_ref, w2e_ref, o_ref):
    t = p_ref[0] + p_ref[1]
    o_ref[...] = lax.dot_general(
        w2e_ref[...], t,
        dimension_numbers=(((0,), (1,)), ((), ())),
        preferred_element_type=jnp.float32,
    )


def _finalize(parts, w2e):
    return pl.pallas_call(
        _fin_body,
        out_shape=jax.ShapeDtypeStruct((OUT_CHANNELS, N_NODES), jnp.float32),
    )(parts, w2e)


def kernel(x, edge_index, edge_attr, W1, b1, W2, b2):
    w1a = W1[:NODE_SIZE]
    w1b = W1[NODE_SIZE:2 * NODE_SIZE]
    w1e = W1[2 * NODE_SIZE:]
    a, b = _compute_ab(x, w1a, w1b)
    e = _compute_e(edge_attr.T.reshape(EDGE_SIZE, 4, _E_ROWS), w1e,
                   jnp.tile(b1, 4).reshape(1, 128))

    idxi3 = edge_index[0].reshape(NW, NCHUNK, CHUNK)
    idxj3 = edge_index[1].reshape(NW, NCHUNK, CHUNK)
    zeros_tile = jnp.zeros((ROWS_PER_TILE, D_SC), jnp.float32)
    hinit = jnp.zeros((CHUNK, D_SC), jnp.float32).at[:, OUT_CHANNELS].set(1.0)

    parts = _sc_scatter(a, b, e, idxi3, idxj3, zeros_tile, hinit)
    parts = parts.reshape(NC, N_NODES, D_SC)

    w2e = jnp.concatenate(
        [W2, b2[None, :], jnp.zeros((D_SC - OUT_CHANNELS - 1, OUT_CHANNELS),
                                    jnp.float32)], axis=0)
    return _finalize(parts, w2e).T


# speedup vs baseline: 8.7208x; 1.3641x over previous
import functools

import jax
import jax.numpy as jnp
from jax import lax
from jax.experimental import pallas as pl
from jax.experimental.pallas import tpu as pltpu
from jax.experimental.pallas import tpu_sc as plsc

N_NODES = 10000
NODE_SIZE = 128
EDGE_SIZE = 16
OUT_CHANNELS = 32
N_EDGES = 320000

NC = 2
NS = 16
NW = NC * NS
E_PER_W = N_EDGES // NW
CHUNK = 80
NCHUNK = E_PER_W // CHUNK
ROWS_PER_TILE = N_NODES // NS
D_SC = 40


def _ab_body(x_ref, wa_ref, wb_ref, a_ref, b_ref):
    xv = x_ref[...]
    a_ref[...] = jnp.dot(xv, wa_ref[...], preferred_element_type=jnp.float32)
    b_ref[...] = jnp.dot(xv, wb_ref[...], preferred_element_type=jnp.float32)


def _compute_ab(x, w1a, w1b):
    return pl.pallas_call(
        _ab_body,
        out_shape=(
            jax.ShapeDtypeStruct((N_NODES, OUT_CHANNELS), jnp.float32),
            jax.ShapeDtypeStruct((N_NODES, OUT_CHANNELS), jnp.float32),
        ),
    )(x, w1a, w1b)


_E_ROWS = N_EDGES // 4
_E_BLK = 16000


def _e_body(eat3_ref, we_ref, b1_ref, e_ref):
    parts = []
    for k in range(4):
        parts.append(lax.dot_general(
            eat3_ref[:, k, :], we_ref[...],
            dimension_numbers=(((0,), (0,)), ((), ())),
            preferred_element_type=jnp.float32,
        ))
    e_ref[...] = jnp.concatenate(parts, axis=1) + b1_ref[...]


def _compute_e(edge_attr_t3, w1e, b1row128):
    grid = (_E_ROWS // _E_BLK,)
    return pl.pallas_call(
        _e_body,
        grid=grid,
        in_specs=[
            pl.BlockSpec((EDGE_SIZE, 4, _E_BLK), lambda i: (0, 0, i)),
            pl.BlockSpec((EDGE_SIZE, OUT_CHANNELS), lambda i: (0, 0)),
            pl.BlockSpec((1, 128), lambda i: (0, 0)),
        ],
        out_specs=pl.BlockSpec((_E_BLK, 128), lambda i: (i, 0)),
        out_shape=jax.ShapeDtypeStruct((_E_ROWS, 128), jnp.float32),
    )(edge_attr_t3, w1e, b1row128)


def _sc_body(a_hbm, b_hbm, e_hbm, idxi_hbm, idxj_hbm, zeros_hbm, hinit_hbm,
             out_hbm, idxi_v, idxj_v, a_buf0, a_buf1, b_buf0, b_buf1,
             e_buf0, e_buf1, h_buf0, h_buf1, stage_v, acc,
             sem_g0, sem_g1, sem_s0, sem_s1):
    c = lax.axis_index("c")
    s = lax.axis_index("s")
    wid = c * NS + s
    a_bufs = (a_buf0, a_buf1)
    b_bufs = (b_buf0, b_buf1)
    e_bufs = (e_buf0, e_buf1)
    h_bufs = (h_buf0, h_buf1)
    sem_g = (sem_g0, sem_g1)
    sem_s = (sem_s0, sem_s1)

    pltpu.sync_copy(idxi_hbm.at[wid], idxi_v)
    pltpu.sync_copy(idxj_hbm.at[wid], idxj_v)
    pltpu.sync_copy(hinit_hbm, h_buf0)
    pltpu.sync_copy(hinit_hbm, h_buf1)
    pltpu.sync_copy(zeros_hbm, stage_v)
    row0 = s * ROWS_PER_TILE
    e_col0 = (wid // 8) * OUT_CHANNELS
    e_row0 = (wid % 8) * E_PER_W

    def start(g, p):
        pltpu.async_copy(a_hbm.at[idxi_v.at[g]], a_bufs[p], sem_g[p])
        pltpu.async_copy(b_hbm.at[idxj_v.at[g]], b_bufs[p], sem_g[p])
        pltpu.async_copy(
            e_hbm.at[pl.ds(e_row0 + g * CHUNK, CHUNK),
                     pl.ds(e_col0, OUT_CHANNELS)],
            e_bufs[p], sem_g[p])

    def drain_gathers(p):
        pltpu.make_async_copy(a_hbm.at[pl.ds(0, CHUNK)], a_bufs[p],
                              sem_g[p]).wait()
        pltpu.make_async_copy(a_hbm.at[pl.ds(0, CHUNK)], b_bufs[p],
                              sem_g[p]).wait()
        pltpu.make_async_copy(
            e_hbm.at[pl.ds(0, CHUNK), pl.ds(0, OUT_CHANNELS)], e_bufs[p],
            sem_g[p]).wait()

    def drain_scatter(p):
        pltpu.make_async_copy(hinit_hbm, h_bufs[p], sem_s[p]).wait()

    def pair_body(gg, carry):
        for p in (0, 1):
            g = 2 * gg + p

            @pl.when(g < NCHUNK)
            def _():
                @pl.when(g + 1 < NCHUNK)
                def _():
                    start(g + 1, 1 - p)

                drain_gathers(p)

                @pl.when(g >= 2)
                def _():
                    drain_scatter(p)

                ab, bb, eb, hb = a_bufs[p], b_bufs[p], e_bufs[p], h_bufs[p]

                def row_body(r, carry2):
                    for hh in (0, 16):
                        sl = pl.ds(hh, 16)
                        hb[r, sl] = jnp.maximum(
                            ab[r, sl] + bb[r, sl] + eb[r, sl], 0.0)
                    return carry2

                lax.fori_loop(0, CHUNK, row_body, 0, unroll=4)
                pltpu.async_copy(hb, acc.at[idxi_v.at[g]], sem_s[p],
                                 add=True)
        return carry

    pltpu.sync_copy(stage_v, acc.at[pl.ds(row0, ROWS_PER_TILE)])
    plsc.subcore_barrier()
    start(0, 0)
    lax.fori_loop(0, (NCHUNK + 1) // 2, pair_body, 0)
    drain_scatter(0)
    drain_scatter(1)
    plsc.subcore_barrier()
    pltpu.sync_copy(acc.at[pl.ds(row0, ROWS_PER_TILE)], stage_v)
    pltpu.sync_copy(stage_v, out_hbm.at[c, s])


def _sc_scatter(a, b, e, idxi3, idxj3, zeros_tile, hinit):
    mesh = plsc.VectorSubcoreMesh(core_axis_name="c", subcore_axis_name="s")
    kfn = pl.kernel(
        _sc_body,
        out_type=jax.ShapeDtypeStruct((NC, NS, ROWS_PER_TILE, D_SC),
                                      jnp.float32),
        mesh=mesh,
        scratch_types=[
            pltpu.VMEM((NCHUNK, CHUNK), jnp.int32),
            pltpu.VMEM((NCHUNK, CHUNK), jnp.int32),
            pltpu.VMEM((CHUNK, OUT_CHANNELS), jnp.float32),
            pltpu.VMEM((CHUNK, OUT_CHANNELS), jnp.float32),
            pltpu.VMEM((CHUNK, OUT_CHANNELS), jnp.float32),
            pltpu.VMEM((CHUNK, OUT_CHANNELS), jnp.float32),
            pltpu.VMEM((CHUNK, OUT_CHANNELS), jnp.float32),
            pltpu.VMEM((CHUNK, OUT_CHANNELS), jnp.float32),
            pltpu.VMEM((CHUNK, D_SC), jnp.float32),
            pltpu.VMEM((CHUNK, D_SC), jnp.float32),
            pltpu.VMEM((ROWS_PER_TILE, D_SC), jnp.float32),
            pltpu.VMEM_SHARED((N_NODES, D_SC), jnp.float32),
            pltpu.SemaphoreType.DMA,
            pltpu.SemaphoreType.DMA,
            pltpu.SemaphoreType.DMA,
            pltpu.SemaphoreType.DMA,
        ],
        compiler_params=pltpu.CompilerParams(use_tc_tiling_on_sc=False),
    )
    return kfn(a, b, e, idxi3, idxj3, zeros_tile, hinit)


def _fin_body(p_ref, w2e_ref, o_ref):
    t = p_ref[0] + p_ref[1]
    o_ref[...] = lax.dot_general(
        w2e_ref[...], t,
        dimension_numbers=(((0,), (1,)), ((), ())),
        preferred_element_type=jnp.float32,
    )


def _finalize(parts, w2e):
    return pl.pallas_call(
        _fin_body,
        out_shape=jax.ShapeDtypeStruct((OUT_CHANNELS, N_NODES), jnp.float32),
    )(parts, w2e)


def kernel(x, edge_index, edge_attr, W1, b1, W2, b2):
    w1a = W1[:NODE_SIZE]
    w1b = W1[NODE_SIZE:2 * NODE_SIZE]
    w1e = W1[2 * NODE_SIZE:]
    a, b = _compute_ab(x, w1a, w1b)
    e = _compute_e(edge_attr.T.reshape(EDGE_SIZE, 4, _E_ROWS), w1e,
                   jnp.tile(b1, 4).reshape(1, 128))

    idxi3 = edge_index[0].reshape(NW, NCHUNK, CHUNK)
    idxj3 = edge_index[1].reshape(NW, NCHUNK, CHUNK)
    zeros_tile = jnp.zeros((ROWS_PER_TILE, D_SC), jnp.float32)
    hinit = jnp.zeros((CHUNK, D_SC), jnp.float32).at[:, OUT_CHANNELS].set(1.0)

    parts = _sc_scatter(a, b, e, idxi3, idxj3, zeros_tile, hinit)
    parts = parts.reshape(NC, N_NODES, D_SC)

    w2e = jnp.concatenate(
        [W2, b2[None, :], jnp.zeros((D_SC - OUT_CHANNELS - 1, OUT_CHANNELS),
                                    jnp.float32)], axis=0)
    return _finalize(parts, w2e).T


# speedup vs baseline: 9.4516x; 1.0838x over previous
import functools

import jax
import jax.numpy as jnp
from jax import lax
from jax.experimental import pallas as pl
from jax.experimental.pallas import tpu as pltpu
from jax.experimental.pallas import tpu_sc as plsc

N_NODES = 10000
NODE_SIZE = 128
EDGE_SIZE = 16
OUT_CHANNELS = 32
N_EDGES = 320000

NC = 2
NS = 16
NW = NC * NS
E_PER_W = N_EDGES // NW
CHUNK = 125
NCHUNK = E_PER_W // CHUNK
ROWS_PER_TILE = N_NODES // NS
D_SC = 40


_E_ROWS = N_EDGES // 4
_E_BLK = 3200
_N4 = N_NODES // 4


def _front_body(x3_ref, wa_ref, wb_ref, eat0_ref, eat1_ref, eat2_ref,
                eat3_ref, we_ref, b1_ref, a4_ref, b4_ref, e_ref):
    @pl.when(pl.program_id(0) == 0)
    def _():
        for w_ref, o_ref in ((wa_ref, a4_ref), (wb_ref, b4_ref)):
            o_ref[...] = jnp.concatenate(
                [jnp.dot(x3_ref[:, k, :], w_ref[...],
                         preferred_element_type=jnp.float32)
                 for k in range(4)], axis=1)

    eats = (eat0_ref, eat1_ref, eat2_ref, eat3_ref)
    e_ref[...] = jnp.concatenate(
        [lax.dot_general(eats[k][...], we_ref[...],
                         dimension_numbers=(((0,), (0,)), ((), ())),
                         preferred_element_type=jnp.float32)
         for k in range(4)], axis=1) + b1_ref[...]


def _compute_front(x3, edge_attr_t, w1a, w1b, w1e, b1row128):
    grid = (_E_ROWS // _E_BLK,)

    def eat_spec(k):
        nblk = _E_ROWS // _E_BLK
        return pl.BlockSpec((EDGE_SIZE, _E_BLK),
                            lambda i, kk=k: (0, nblk * kk + i))

    return pl.pallas_call(
        _front_body,
        grid=grid,
        in_specs=[
            pl.BlockSpec((_N4, 4, NODE_SIZE), lambda i: (0, 0, 0)),
            pl.BlockSpec((NODE_SIZE, OUT_CHANNELS), lambda i: (0, 0)),
            pl.BlockSpec((NODE_SIZE, OUT_CHANNELS), lambda i: (0, 0)),
            eat_spec(0), eat_spec(1), eat_spec(2), eat_spec(3),
            pl.BlockSpec((EDGE_SIZE, OUT_CHANNELS), lambda i: (0, 0)),
            pl.BlockSpec((1, 128), lambda i: (0, 0)),
        ],
        out_specs=(
            pl.BlockSpec((_N4, 128), lambda i: (0, 0)),
            pl.BlockSpec((_N4, 128), lambda i: (0, 0)),
            pl.BlockSpec((_E_BLK, 128), lambda i: (i, 0)),
        ),
        out_shape=(
            jax.ShapeDtypeStruct((_N4, 128), jnp.float32),
            jax.ShapeDtypeStruct((_N4, 128), jnp.float32),
            jax.ShapeDtypeStruct((_E_ROWS, 128), jnp.float32),
        ),
    )(x3, w1a, w1b, edge_attr_t, edge_attr_t, edge_attr_t, edge_attr_t,
      w1e, b1row128)


def _sc_body(a_hbm, b_hbm, e_hbm, idxi_hbm, idxj_hbm, zeros_hbm, hinit_hbm,
             out_hbm, idxi_v, idxj_v, a_buf0, a_buf1, b_buf0, b_buf1,
             e_buf0, e_buf1, h_buf0, h_buf1, stage_v, acc,
             sem_g0, sem_g1, sem_s0, sem_s1):
    c = lax.axis_index("c")
    s = lax.axis_index("s")
    wid = c * NS + s
    a_bufs = (a_buf0, a_buf1)
    b_bufs = (b_buf0, b_buf1)
    e_bufs = (e_buf0, e_buf1)
    h_bufs = (h_buf0, h_buf1)
    sem_g = (sem_g0, sem_g1)
    sem_s = (sem_s0, sem_s1)

    pltpu.sync_copy(idxi_hbm.at[wid], idxi_v)
    pltpu.sync_copy(idxj_hbm.at[wid], idxj_v)
    pltpu.sync_copy(hinit_hbm, h_buf0)
    pltpu.sync_copy(hinit_hbm, h_buf1)
    pltpu.sync_copy(zeros_hbm, stage_v)
    row0 = s * ROWS_PER_TILE
    e_col0 = (wid // 8) * OUT_CHANNELS
    e_row0 = (wid % 8) * E_PER_W

    def start(g, p):
        pltpu.async_copy(a_hbm.at[idxi_v.at[g]], a_bufs[p], sem_g[p])
        pltpu.async_copy(b_hbm.at[idxj_v.at[g]], b_bufs[p], sem_g[p])
        pltpu.async_copy(
            e_hbm.at[pl.ds(e_row0 + g * CHUNK, CHUNK),
                     pl.ds(e_col0, OUT_CHANNELS)],
            e_bufs[p], sem_g[p])

    def drain_gathers(p):
        pltpu.make_async_copy(a_hbm.at[pl.ds(0, CHUNK)], a_bufs[p],
                              sem_g[p]).wait()
        pltpu.make_async_copy(a_hbm.at[pl.ds(0, CHUNK)], b_bufs[p],
                              sem_g[p]).wait()
        pltpu.make_async_copy(
            e_hbm.at[pl.ds(0, CHUNK), pl.ds(0, OUT_CHANNELS)], e_bufs[p],
            sem_g[p]).wait()

    def drain_scatter(p):
        pltpu.make_async_copy(hinit_hbm, h_bufs[p], sem_s[p]).wait()

    def pair_body(gg, carry):
        for p in (0, 1):
            g = 2 * gg + p

            @pl.when(g < NCHUNK)
            def _():
                @pl.when(g + 1 < NCHUNK)
                def _():
                    start(g + 1, 1 - p)

                drain_gathers(p)

                @pl.when(g >= 2)
                def _():
                    drain_scatter(p)

                ab, bb, eb, hb = a_bufs[p], b_bufs[p], e_bufs[p], h_bufs[p]

                def row_body(r, carry2):
                    for hh in (0, 16):
                        sl = pl.ds(hh, 16)
                        hb[r, sl] = jnp.maximum(
                            ab[r, sl] + bb[r, sl] + eb[r, sl], 0.0)
                    return carry2

                lax.fori_loop(0, CHUNK, row_body, 0, unroll=4)
                pltpu.async_copy(hb, acc.at[idxi_v.at[g]], sem_s[p],
                                 add=True)
        return carry

    pltpu.sync_copy(stage_v, acc.at[pl.ds(row0, ROWS_PER_TILE)])
    plsc.subcore_barrier()
    start(0, 0)
    lax.fori_loop(0, (NCHUNK + 1) // 2, pair_body, 0)
    drain_scatter(0)
    drain_scatter(1)
    plsc.subcore_barrier()
    pltpu.sync_copy(acc.at[pl.ds(row0, ROWS_PER_TILE)], stage_v)
    pltpu.sync_copy(stage_v, out_hbm.at[c, s])


def _sc_scatter(a, b, e, idxi3, idxj3, zeros_tile, hinit):
    mesh = plsc.VectorSubcoreMesh(core_axis_name="c", subcore_axis_name="s")
    kfn = pl.kernel(
        _sc_body,
        out_type=jax.ShapeDtypeStruct((NC, NS, ROWS_PER_TILE, D_SC),
                                      jnp.float32),
        mesh=mesh,
        scratch_types=[
            pltpu.VMEM((NCHUNK, CHUNK), jnp.int32),
            pltpu.VMEM((NCHUNK, CHUNK), jnp.int32),
            pltpu.VMEM((CHUNK, OUT_CHANNELS), jnp.float32),
            pltpu.VMEM((CHUNK, OUT_CHANNELS), jnp.float32),
            pltpu.VMEM((CHUNK, OUT_CHANNELS), jnp.float32),
            pltpu.VMEM((CHUNK, OUT_CHANNELS), jnp.float32),
            pltpu.VMEM((CHUNK, OUT_CHANNELS), jnp.float32),
            pltpu.VMEM((CHUNK, OUT_CHANNELS), jnp.float32),
            pltpu.VMEM((CHUNK, D_SC), jnp.float32),
            pltpu.VMEM((CHUNK, D_SC), jnp.float32),
            pltpu.VMEM((ROWS_PER_TILE, D_SC), jnp.float32),
            pltpu.VMEM_SHARED((N_NODES, D_SC), jnp.float32),
            pltpu.SemaphoreType.DMA,
            pltpu.SemaphoreType.DMA,
            pltpu.SemaphoreType.DMA,
            pltpu.SemaphoreType.DMA,
        ],
        compiler_params=pltpu.CompilerParams(use_tc_tiling_on_sc=False),
    )
    return kfn(a, b, e, idxi3, idxj3, zeros_tile, hinit)


def _fin_body(p_ref, w2e_ref, o_ref):
    t = p_ref[0] + p_ref[1]
    o_ref[...] = lax.dot_general(
        w2e_ref[...], t,
        dimension_numbers=(((0,), (1,)), ((), ())),
        preferred_element_type=jnp.float32,
    )


def _finalize(parts, w2e):
    return pl.pallas_call(
        _fin_body,
        out_shape=jax.ShapeDtypeStruct((OUT_CHANNELS, N_NODES), jnp.float32),
    )(parts, w2e)


def kernel(x, edge_index, edge_attr, W1, b1, W2, b2):
    w1a = W1[:NODE_SIZE]
    w1b = W1[NODE_SIZE:2 * NODE_SIZE]
    w1e = W1[2 * NODE_SIZE:]
    a4, b4, e = _compute_front(
        x.reshape(_N4, 4, NODE_SIZE), edge_attr.T, w1a, w1b, w1e,
        jnp.tile(b1, 4).reshape(1, 128))
    a = a4.reshape(N_NODES, OUT_CHANNELS)
    b = b4.reshape(N_NODES, OUT_CHANNELS)

    idxi3 = edge_index[0].reshape(NW, NCHUNK, CHUNK)
    idxj3 = edge_index[1].reshape(NW, NCHUNK, CHUNK)
    zeros_tile = jnp.zeros((ROWS_PER_TILE, D_SC), jnp.float32)
    hinit = jnp.zeros((CHUNK, D_SC), jnp.float32).at[:, OUT_CHANNELS].set(1.0)

    parts = _sc_scatter(a, b, e, idxi3, idxj3, zeros_tile, hinit)
    parts = parts.reshape(NC, N_NODES, D_SC)

    w2e = jnp.concatenate(
        [W2, b2[None, :], jnp.zeros((D_SC - OUT_CHANNELS - 1, OUT_CHANNELS),
                                    jnp.float32)], axis=0)
    return _finalize(parts, w2e).T


# speedup vs baseline: 11.0119x; 1.1651x over previous
import functools

import jax
import jax.numpy as jnp
from jax import lax
from jax.experimental import pallas as pl
from jax.experimental.pallas import tpu as pltpu
from jax.experimental.pallas import tpu_sc as plsc

N_NODES = 10000
NODE_SIZE = 128
EDGE_SIZE = 16
OUT_CHANNELS = 32
N_EDGES = 320000

NC = 2
NS = 16
NW = NC * NS
E_PER_W = N_EDGES // NW
CHUNK = 125
NCHUNK = E_PER_W // CHUNK
ROWS_PER_TILE = N_NODES // NS
D_SC = 40


_E_ROWS = N_EDGES // 4
_E_BLK = 3200
_N4 = N_NODES // 4


def _front_body(x4_ref, wa4_ref, wb4_ref, eat0_ref, eat1_ref, eat2_ref,
                eat3_ref, we4_ref, b1_ref, a4_ref, b4_ref, e_ref):
    @pl.when(pl.program_id(0) == 0)
    def _():
        xv = x4_ref[...]
        a4_ref[...] = jnp.dot(xv, wa4_ref[...],
                              preferred_element_type=jnp.float32)
        b4_ref[...] = jnp.dot(xv, wb4_ref[...],
                              preferred_element_type=jnp.float32)

    cat = jnp.concatenate(
        [eat0_ref[...], eat1_ref[...], eat2_ref[...], eat3_ref[...]], axis=0)
    e_ref[...] = lax.dot_general(
        cat, we4_ref[...],
        dimension_numbers=(((0,), (0,)), ((), ())),
        preferred_element_type=jnp.float32,
    ) + b1_ref[...]


def _compute_front(x4, edge_attr_t, wa4, wb4, we4, b1row128):
    grid = (_E_ROWS // _E_BLK,)

    def eat_spec(k):
        nblk = _E_ROWS // _E_BLK
        return pl.BlockSpec((EDGE_SIZE, _E_BLK),
                            lambda i, kk=k: (0, nblk * kk + i))

    return pl.pallas_call(
        _front_body,
        grid=grid,
        in_specs=[
            pl.BlockSpec((_N4, 4 * NODE_SIZE), lambda i: (0, 0)),
            pl.BlockSpec((4 * NODE_SIZE, 128), lambda i: (0, 0)),
            pl.BlockSpec((4 * NODE_SIZE, 128), lambda i: (0, 0)),
            eat_spec(0), eat_spec(1), eat_spec(2), eat_spec(3),
            pl.BlockSpec((4 * EDGE_SIZE, 128), lambda i: (0, 0)),
            pl.BlockSpec((1, 128), lambda i: (0, 0)),
        ],
        out_specs=(
            pl.BlockSpec((_N4, 128), lambda i: (0, 0)),
            pl.BlockSpec((_N4, 128), lambda i: (0, 0)),
            pl.BlockSpec((_E_BLK, 128), lambda i: (i, 0)),
        ),
        out_shape=(
            jax.ShapeDtypeStruct((_N4, 128), jnp.float32),
            jax.ShapeDtypeStruct((_N4, 128), jnp.float32),
            jax.ShapeDtypeStruct((_E_ROWS, 128), jnp.float32),
        ),
    )(x4, wa4, wb4, edge_attr_t, edge_attr_t, edge_attr_t, edge_attr_t,
      we4, b1row128)


def _sc_body(a_hbm, b_hbm, e_hbm, idxi_hbm, idxj_hbm, zeros_hbm, hinit_hbm,
             out_hbm, idxi_v, idxj_v, a_buf0, a_buf1, b_buf0, b_buf1,
             e_buf0, e_buf1, h_buf0, h_buf1, stage_v, acc,
             sem_g0, sem_g1, sem_s0, sem_s1):
    c = lax.axis_index("c")
    s = lax.axis_index("s")
    wid = c * NS + s
    a_bufs = (a_buf0, a_buf1)
    b_bufs = (b_buf0, b_buf1)
    e_bufs = (e_buf0, e_buf1)
    h_bufs = (h_buf0, h_buf1)
    sem_g = (sem_g0, sem_g1)
    sem_s = (sem_s0, sem_s1)

    pltpu.sync_copy(idxi_hbm.at[wid], idxi_v)
    pltpu.sync_copy(idxj_hbm.at[wid], idxj_v)
    pltpu.sync_copy(hinit_hbm, h_buf0)
    pltpu.sync_copy(hinit_hbm, h_buf1)
    pltpu.sync_copy(zeros_hbm, stage_v)
    row0 = s * ROWS_PER_TILE
    e_col0 = (wid // 8) * OUT_CHANNELS
    e_row0 = (wid % 8) * E_PER_W

    def start(g, p):
        pltpu.async_copy(a_hbm.at[idxi_v.at[g]], a_bufs[p], sem_g[p])
        pltpu.async_copy(b_hbm.at[idxj_v.at[g]], b_bufs[p], sem_g[p])
        pltpu.async_copy(
            e_hbm.at[pl.ds(e_row0 + g * CHUNK, CHUNK),
                     pl.ds(e_col0, OUT_CHANNELS)],
            e_bufs[p], sem_g[p])

    def drain_gathers(p):
        pltpu.make_async_copy(a_hbm.at[pl.ds(0, CHUNK)], a_bufs[p],
                              sem_g[p]).wait()
        pltpu.make_async_copy(a_hbm.at[pl.ds(0, CHUNK)], b_bufs[p],
                              sem_g[p]).wait()
        pltpu.make_async_copy(
            e_hbm.at[pl.ds(0, CHUNK), pl.ds(0, OUT_CHANNELS)], e_bufs[p],
            sem_g[p]).wait()

    def drain_scatter(p):
        pltpu.make_async_copy(hinit_hbm, h_bufs[p], sem_s[p]).wait()

    def pair_body(gg, carry):
        for p in (0, 1):
            g = 2 * gg + p

            @pl.when(g < NCHUNK)
            def _():
                @pl.when(g + 1 < NCHUNK)
                def _():
                    start(g + 1, 1 - p)

                drain_gathers(p)

                @pl.when(g >= 2)
                def _():
                    drain_scatter(p)

                ab, bb, eb, hb = a_bufs[p], b_bufs[p], e_bufs[p], h_bufs[p]

                def row_body(r, carry2):
                    for hh in (0, 16):
                        sl = pl.ds(hh, 16)
                        hb[r, sl] = jnp.maximum(
                            ab[r, sl] + bb[r, sl] + eb[r, sl], 0.0)
                    return carry2

                lax.fori_loop(0, CHUNK, row_body, 0, unroll=4)
                pltpu.async_copy(hb, acc.at[idxi_v.at[g]], sem_s[p],
                                 add=True)
        return carry

    pltpu.sync_copy(stage_v, acc.at[pl.ds(row0, ROWS_PER_TILE)])
    plsc.subcore_barrier()
    start(0, 0)
    lax.fori_loop(0, (NCHUNK + 1) // 2, pair_body, 0)
    drain_scatter(0)
    drain_scatter(1)
    plsc.subcore_barrier()
    pltpu.sync_copy(acc.at[pl.ds(row0, ROWS_PER_TILE)], stage_v)
    pltpu.sync_copy(stage_v, out_hbm.at[c, s])


def _sc_scatter(a, b, e, idxi3, idxj3, zeros_tile, hinit):
    mesh = plsc.VectorSubcoreMesh(core_axis_name="c", subcore_axis_name="s")
    kfn = pl.kernel(
        _sc_body,
        out_type=jax.ShapeDtypeStruct((NC, NS, ROWS_PER_TILE, D_SC),
                                      jnp.float32),
        mesh=mesh,
        scratch_types=[
            pltpu.VMEM((NCHUNK, CHUNK), jnp.int32),
            pltpu.VMEM((NCHUNK, CHUNK), jnp.int32),
            pltpu.VMEM((CHUNK, OUT_CHANNELS), jnp.float32),
            pltpu.VMEM((CHUNK, OUT_CHANNELS), jnp.float32),
            pltpu.VMEM((CHUNK, OUT_CHANNELS), jnp.float32),
            pltpu.VMEM((CHUNK, OUT_CHANNELS), jnp.float32),
            pltpu.VMEM((CHUNK, OUT_CHANNELS), jnp.float32),
            pltpu.VMEM((CHUNK, OUT_CHANNELS), jnp.float32),
            pltpu.VMEM((CHUNK, D_SC), jnp.float32),
            pltpu.VMEM((CHUNK, D_SC), jnp.float32),
            pltpu.VMEM((ROWS_PER_TILE, D_SC), jnp.float32),
            pltpu.VMEM_SHARED((N_NODES, D_SC), jnp.float32),
            pltpu.SemaphoreType.DMA,
            pltpu.SemaphoreType.DMA,
            pltpu.SemaphoreType.DMA,
            pltpu.SemaphoreType.DMA,
        ],
        compiler_params=pltpu.CompilerParams(use_tc_tiling_on_sc=False),
    )
    return kfn(a, b, e, idxi3, idxj3, zeros_tile, hinit)


def _fin_body(p_ref, w2e_ref, o_ref):
    t = p_ref[0] + p_ref[1]
    o_ref[...] = lax.dot_general(
        w2e_ref[...], t,
        dimension_numbers=(((0,), (1,)), ((), ())),
        preferred_element_type=jnp.float32,
    )


def _finalize(parts, w2e):
    return pl.pallas_call(
        _fin_body,
        out_shape=jax.ShapeDtypeStruct((OUT_CHANNELS, N_NODES), jnp.float32),
    )(parts, w2e)


def kernel(x, edge_index, edge_attr, W1, b1, W2, b2):
    w1a = W1[:NODE_SIZE]
    w1b = W1[NODE_SIZE:2 * NODE_SIZE]
    w1e = W1[2 * NODE_SIZE:]
    eye4 = jnp.eye(4, dtype=jnp.float32)
    a4, b4, e = _compute_front(
        x.reshape(_N4, 4 * NODE_SIZE), edge_attr.T,
        jnp.kron(eye4, w1a), jnp.kron(eye4, w1b), jnp.kron(eye4, w1e),
        jnp.tile(b1, 4).reshape(1, 128))
    a = a4.reshape(N_NODES, OUT_CHANNELS)
    b = b4.reshape(N_NODES, OUT_CHANNELS)

    idxi3 = edge_index[0].reshape(NW, NCHUNK, CHUNK)
    idxj3 = edge_index[1].reshape(NW, NCHUNK, CHUNK)
    zeros_tile = jnp.zeros((ROWS_PER_TILE, D_SC), jnp.float32)
    hinit = jnp.zeros((CHUNK, D_SC), jnp.float32).at[:, OUT_CHANNELS].set(1.0)

    parts = _sc_scatter(a, b, e, idxi3, idxj3, zeros_tile, hinit)
    parts = parts.reshape(NC, N_NODES, D_SC)

    w2e = jnp.concatenate(
        [W2, b2[None, :], jnp.zeros((D_SC - OUT_CHANNELS - 1, OUT_CHANNELS),
                                    jnp.float32)], axis=0)
    return _finalize(parts, w2e).T


# speedup vs baseline: 11.4655x; 1.0412x over previous
import functools

import jax
import jax.numpy as jnp
from jax import lax
from jax.experimental import pallas as pl
from jax.experimental.pallas import tpu as pltpu
from jax.experimental.pallas import tpu_sc as plsc

N_NODES = 10000
NODE_SIZE = 128
EDGE_SIZE = 16
OUT_CHANNELS = 32
N_EDGES = 320000

NC = 2
NS = 16
NW = NC * NS
E_PER_W = N_EDGES // NW
CHUNK = 125
NCHUNK = E_PER_W // CHUNK
ROWS_PER_TILE = N_NODES // NS
D_SC = 40


_E_ROWS = N_EDGES // 4
_E_BLK = 16000
_N4 = N_NODES // 4


def _front_body(x4_ref, wa4_ref, wb4_ref, eat0_ref, eat1_ref, eat2_ref,
                eat3_ref, we4_ref, b1_ref, a4_ref, b4_ref, e_ref):
    @pl.when(pl.program_id(0) == 0)
    def _():
        xv = x4_ref[...]
        a4_ref[...] = jnp.dot(xv, wa4_ref[...],
                              preferred_element_type=jnp.float32)
        b4_ref[...] = jnp.dot(xv, wb4_ref[...],
                              preferred_element_type=jnp.float32)

    cat = jnp.concatenate(
        [eat0_ref[...], eat1_ref[...], eat2_ref[...], eat3_ref[...]], axis=0)
    e_ref[...] = lax.dot_general(
        cat, we4_ref[...],
        dimension_numbers=(((0,), (0,)), ((), ())),
        preferred_element_type=jnp.float32,
    ) + b1_ref[...]


def _compute_front(x4, edge_attr_t, wa4, wb4, we4, b1row128):
    grid = (_E_ROWS // _E_BLK,)

    def eat_spec(k):
        nblk = _E_ROWS // _E_BLK
        return pl.BlockSpec((EDGE_SIZE, _E_BLK),
                            lambda i, kk=k: (0, nblk * kk + i))

    return pl.pallas_call(
        _front_body,
        grid=grid,
        in_specs=[
            pl.BlockSpec((_N4, 4 * NODE_SIZE), lambda i: (0, 0)),
            pl.BlockSpec((4 * NODE_SIZE, 128), lambda i: (0, 0)),
            pl.BlockSpec((4 * NODE_SIZE, 128), lambda i: (0, 0)),
            eat_spec(0), eat_spec(1), eat_spec(2), eat_spec(3),
            pl.BlockSpec((4 * EDGE_SIZE, 128), lambda i: (0, 0)),
            pl.BlockSpec((1, 128), lambda i: (0, 0)),
        ],
        out_specs=(
            pl.BlockSpec((_N4, 128), lambda i: (0, 0)),
            pl.BlockSpec((_N4, 128), lambda i: (0, 0)),
            pl.BlockSpec((_E_BLK, 128), lambda i: (i, 0)),
        ),
        out_shape=(
            jax.ShapeDtypeStruct((_N4, 128), jnp.float32),
            jax.ShapeDtypeStruct((_N4, 128), jnp.float32),
            jax.ShapeDtypeStruct((_E_ROWS, 128), jnp.float32),
        ),
    )(x4, wa4, wb4, edge_attr_t, edge_attr_t, edge_attr_t, edge_attr_t,
      we4, b1row128)


def _sc_body(a_hbm, b_hbm, e_hbm, idxi_hbm, idxj_hbm, zeros_hbm, hinit_hbm,
             out_hbm, idxi_v, idxj_v, a_buf0, a_buf1, b_buf0, b_buf1,
             e_buf0, e_buf1, h_buf0, h_buf1, stage_v, acc,
             sem_g0, sem_g1, sem_s0, sem_s1):
    c = lax.axis_index("c")
    s = lax.axis_index("s")
    wid = c * NS + s
    a_bufs = (a_buf0, a_buf1)
    b_bufs = (b_buf0, b_buf1)
    e_bufs = (e_buf0, e_buf1)
    h_bufs = (h_buf0, h_buf1)
    sem_g = (sem_g0, sem_g1)
    sem_s = (sem_s0, sem_s1)

    pltpu.sync_copy(idxi_hbm.at[wid], idxi_v)
    pltpu.sync_copy(idxj_hbm.at[wid], idxj_v)
    pltpu.sync_copy(hinit_hbm, h_buf0)
    pltpu.sync_copy(hinit_hbm, h_buf1)
    pltpu.sync_copy(zeros_hbm, stage_v)
    row0 = s * ROWS_PER_TILE
    e_col0 = (wid // 8) * OUT_CHANNELS
    e_row0 = (wid % 8) * E_PER_W

    def start(g, p):
        pltpu.async_copy(a_hbm.at[idxi_v.at[g]], a_bufs[p], sem_g[p])
        pltpu.async_copy(b_hbm.at[idxj_v.at[g]], b_bufs[p], sem_g[p])
        pltpu.async_copy(
            e_hbm.at[pl.ds(e_row0 + g * CHUNK, CHUNK),
                     pl.ds(e_col0, OUT_CHANNELS)],
            e_bufs[p], sem_g[p])

    def drain_gathers(p):
        pltpu.make_async_copy(a_hbm.at[pl.ds(0, CHUNK)], a_bufs[p],
                              sem_g[p]).wait()
        pltpu.make_async_copy(a_hbm.at[pl.ds(0, CHUNK)], b_bufs[p],
                              sem_g[p]).wait()
        pltpu.make_async_copy(
            e_hbm.at[pl.ds(0, CHUNK), pl.ds(0, OUT_CHANNELS)], e_bufs[p],
            sem_g[p]).wait()

    def drain_scatter(p):
        pltpu.make_async_copy(hinit_hbm, h_bufs[p], sem_s[p]).wait()

    def pair_body(gg, carry):
        for p in (0, 1):
            g = 2 * gg + p

            @pl.when(g < NCHUNK)
            def _():
                @pl.when(g + 1 < NCHUNK)
                def _():
                    start(g + 1, 1 - p)

                drain_gathers(p)

                @pl.when(g >= 2)
                def _():
                    drain_scatter(p)

                ab, bb, eb, hb = a_bufs[p], b_bufs[p], e_bufs[p], h_bufs[p]

                def row_body(r, carry2):
                    for hh in (0, 16):
                        sl = pl.ds(hh, 16)
                        hb[r, sl] = jnp.maximum(
                            ab[r, sl] + bb[r, sl] + eb[r, sl], 0.0)
                    return carry2

                lax.fori_loop(0, CHUNK, row_body, 0, unroll=4)
                pltpu.async_copy(hb, acc.at[idxi_v.at[g]], sem_s[p],
                                 add=True)
        return carry

    pltpu.sync_copy(stage_v, acc.at[pl.ds(row0, ROWS_PER_TILE)])
    plsc.subcore_barrier()
    start(0, 0)
    lax.fori_loop(0, (NCHUNK + 1) // 2, pair_body, 0)
    drain_scatter(0)
    drain_scatter(1)
    plsc.subcore_barrier()
    pltpu.sync_copy(acc.at[pl.ds(row0, ROWS_PER_TILE)], stage_v)
    pltpu.sync_copy(stage_v, out_hbm.at[c, s])


def _sc_scatter(a, b, e, idxi3, idxj3, zeros_tile, hinit):
    mesh = plsc.VectorSubcoreMesh(core_axis_name="c", subcore_axis_name="s")
    kfn = pl.kernel(
        _sc_body,
        out_type=jax.ShapeDtypeStruct((NC, NS, ROWS_PER_TILE, D_SC),
                                      jnp.float32),
        mesh=mesh,
        scratch_types=[
            pltpu.VMEM((NCHUNK, CHUNK), jnp.int32),
            pltpu.VMEM((NCHUNK, CHUNK), jnp.int32),
            pltpu.VMEM((CHUNK, OUT_CHANNELS), jnp.float32),
            pltpu.VMEM((CHUNK, OUT_CHANNELS), jnp.float32),
            pltpu.VMEM((CHUNK, OUT_CHANNELS), jnp.float32),
            pltpu.VMEM((CHUNK, OUT_CHANNELS), jnp.float32),
            pltpu.VMEM((CHUNK, OUT_CHANNELS), jnp.float32),
            pltpu.VMEM((CHUNK, OUT_CHANNELS), jnp.float32),
            pltpu.VMEM((CHUNK, D_SC), jnp.float32),
            pltpu.VMEM((CHUNK, D_SC), jnp.float32),
            pltpu.VMEM((ROWS_PER_TILE, D_SC), jnp.float32),
            pltpu.VMEM_SHARED((N_NODES, D_SC), jnp.float32),
            pltpu.SemaphoreType.DMA,
            pltpu.SemaphoreType.DMA,
            pltpu.SemaphoreType.DMA,
            pltpu.SemaphoreType.DMA,
        ],
        compiler_params=pltpu.CompilerParams(use_tc_tiling_on_sc=False),
    )
    return kfn(a, b, e, idxi3, idxj3, zeros_tile, hinit)


def _fin_body(p_ref, w2e_ref, o_ref):
    t = p_ref[0] + p_ref[1]
    o_ref[...] = lax.dot_general(
        w2e_ref[...], t,
        dimension_numbers=(((0,), (1,)), ((), ())),
        preferred_element_type=jnp.float32,
    )


def _finalize(parts, w2e):
    return pl.pallas_call(
        _fin_body,
        out_shape=jax.ShapeDtypeStruct((OUT_CHANNELS, N_NODES), jnp.float32),
    )(parts, w2e)


def kernel(x, edge_index, edge_attr, W1, b1, W2, b2):
    w1a = W1[:NODE_SIZE]
    w1b = W1[NODE_SIZE:2 * NODE_SIZE]
    w1e = W1[2 * NODE_SIZE:]
    eye4 = jnp.eye(4, dtype=jnp.float32)
    a4, b4, e = _compute_front(
        x.reshape(_N4, 4 * NODE_SIZE), edge_attr.T,
        jnp.kron(eye4, w1a), jnp.kron(eye4, w1b), jnp.kron(eye4, w1e),
        jnp.tile(b1, 4).reshape(1, 128))
    a = a4.reshape(N_NODES, OUT_CHANNELS)
    b = b4.reshape(N_NODES, OUT_CHANNELS)

    idxi3 = edge_index[0].reshape(NW, NCHUNK, CHUNK)
    idxj3 = edge_index[1].reshape(NW, NCHUNK, CHUNK)
    zeros_tile = jnp.zeros((ROWS_PER_TILE, D_SC), jnp.float32)
    hinit = jnp.zeros((CHUNK, D_SC), jnp.float32).at[:, OUT_CHANNELS].set(1.0)

    parts = _sc_scatter(a, b, e, idxi3, idxj3, zeros_tile, hinit)
    parts = parts.reshape(NC, N_NODES, D_SC)

    w2e = jnp.concatenate(
        [W2, b2[None, :], jnp.zeros((D_SC - OUT_CHANNELS - 1, OUT_CHANNELS),
                                    jnp.float32)], axis=0)
    return _finalize(parts, w2e).T


# speedup vs baseline: 16.0786x; 1.4023x over previous
import functools

import jax
import jax.numpy as jnp
from jax import lax
from jax.experimental import pallas as pl
from jax.experimental.pallas import tpu as pltpu
from jax.experimental.pallas import tpu_sc as plsc

N_NODES = 10000
NODE_SIZE = 128
EDGE_SIZE = 16
OUT_CHANNELS = 32
N_EDGES = 320000

NC = 2
NS = 16
NW = NC * NS
E_PER_W = N_EDGES // NW
CHUNK = 125
NCHUNK = E_PER_W // CHUNK
ROWS_PER_TILE = N_NODES // NS
D_SC = 40


_E_ROWS = N_EDGES // 4
_E_BLK = 16000
_N4 = N_NODES // 4


def _front_body(x4_ref, wa4_ref, wb4_ref, eat0_ref, eat1_ref, eat2_ref,
                eat3_ref, we4_ref, b1_ref, a4_ref, b4_ref, e_ref):
    @pl.when(pl.program_id(0) == 0)
    def _():
        xv = x4_ref[...]
        a4_ref[...] = jnp.dot(xv, wa4_ref[...],
                              preferred_element_type=jnp.float32)
        b4_ref[...] = jnp.dot(xv, wb4_ref[...],
                              preferred_element_type=jnp.float32)

    cat = jnp.concatenate(
        [eat0_ref[...], eat1_ref[...], eat2_ref[...], eat3_ref[...]], axis=0)
    e_ref[...] = lax.dot_general(
        cat, we4_ref[...],
        dimension_numbers=(((0,), (0,)), ((), ())),
        preferred_element_type=jnp.float32,
    ) + b1_ref[...]


def _compute_front(x4, edge_attr_t, wa4, wb4, we4, b1row128):
    grid = (_E_ROWS // _E_BLK,)

    def eat_spec(k):
        nblk = _E_ROWS // _E_BLK
        return pl.BlockSpec((EDGE_SIZE, _E_BLK),
                            lambda i, kk=k: (0, nblk * kk + i))

    return pl.pallas_call(
        _front_body,
        grid=grid,
        in_specs=[
            pl.BlockSpec((_N4, 4 * NODE_SIZE), lambda i: (0, 0)),
            pl.BlockSpec((4 * NODE_SIZE, 128), lambda i: (0, 0)),
            pl.BlockSpec((4 * NODE_SIZE, 128), lambda i: (0, 0)),
            eat_spec(0), eat_spec(1), eat_spec(2), eat_spec(3),
            pl.BlockSpec((4 * EDGE_SIZE, 128), lambda i: (0, 0)),
            pl.BlockSpec((1, 128), lambda i: (0, 0)),
        ],
        out_specs=(
            pl.BlockSpec((_N4, 128), lambda i: (0, 0)),
            pl.BlockSpec((_N4, 128), lambda i: (0, 0)),
            pl.BlockSpec((_E_BLK, 128), lambda i: (i, 0)),
        ),
        out_shape=(
            jax.ShapeDtypeStruct((_N4, 128), jnp.float32),
            jax.ShapeDtypeStruct((_N4, 128), jnp.float32),
            jax.ShapeDtypeStruct((_E_ROWS, 128), jnp.float32),
        ),
    )(x4, wa4, wb4, edge_attr_t, edge_attr_t, edge_attr_t, edge_attr_t,
      we4, b1row128)


def _sc_body(a_hbm, b_hbm, e_hbm, idxi_hbm, idxj_hbm, zeros_hbm, hinit_hbm,
             out_hbm, idxi_v, idxj_v, a_buf0, a_buf1, b_buf0, b_buf1,
             e_buf0, e_buf1, h_buf0, h_buf1, stage_v, acc,
             sem_g0, sem_g1, sem_s0, sem_s1):
    c = lax.axis_index("c")
    s = lax.axis_index("s")
    wid = c * NS + s
    a_bufs = (a_buf0, a_buf1)
    b_bufs = (b_buf0, b_buf1)
    e_bufs = (e_buf0, e_buf1)
    h_bufs = (h_buf0, h_buf1)
    sem_g = (sem_g0, sem_g1)
    sem_s = (sem_s0, sem_s1)

    pltpu.sync_copy(idxi_hbm.at[wid], idxi_v)
    pltpu.sync_copy(idxj_hbm.at[wid], idxj_v)
    pltpu.sync_copy(hinit_hbm, h_buf0)
    pltpu.sync_copy(hinit_hbm, h_buf1)
    pltpu.sync_copy(zeros_hbm, stage_v)
    row0 = s * ROWS_PER_TILE
    e_col0 = (wid // 8) * OUT_CHANNELS
    e_row0 = (wid % 8) * E_PER_W

    def start(g, p):
        pltpu.async_copy(a_hbm.at[idxi_v.at[g]], a_bufs[p], sem_g[p])
        pltpu.async_copy(b_hbm.at[idxj_v.at[g]], b_bufs[p], sem_g[p])
        pltpu.async_copy(
            e_hbm.at[pl.ds(e_row0 + g * CHUNK, CHUNK),
                     pl.ds(e_col0, OUT_CHANNELS)],
            e_bufs[p], sem_g[p])

    def drain_gathers(p):
        pltpu.make_async_copy(a_hbm.at[pl.ds(0, CHUNK)], a_bufs[p],
                              sem_g[p]).wait()
        pltpu.make_async_copy(a_hbm.at[pl.ds(0, CHUNK)], b_bufs[p],
                              sem_g[p]).wait()
        pltpu.make_async_copy(
            e_hbm.at[pl.ds(0, CHUNK), pl.ds(0, OUT_CHANNELS)], e_bufs[p],
            sem_g[p]).wait()

    def drain_scatter(p):
        pltpu.make_async_copy(hinit_hbm, h_bufs[p], sem_s[p]).wait()

    def pair_body(gg, carry):
        for p in (0, 1):
            g = 2 * gg + p

            @pl.when(g < NCHUNK)
            def _():
                @pl.when(g + 1 < NCHUNK)
                def _():
                    start(g + 1, 1 - p)

                drain_gathers(p)

                @pl.when(g >= 2)
                def _():
                    drain_scatter(p)

                ab, bb, eb, hb = a_bufs[p], b_bufs[p], e_bufs[p], h_bufs[p]

                @plsc.parallel_loop(0, CHUNK, unroll=5)
                def _(r):
                    for hh in (0, 16):
                        sl = pl.ds(hh, 16)
                        hb[r, sl] = jnp.maximum(
                            ab[r, sl] + bb[r, sl] + eb[r, sl], 0.0)
                pltpu.async_copy(hb, acc.at[idxi_v.at[g]], sem_s[p],
                                 add=True)
        return carry

    pltpu.sync_copy(stage_v, acc.at[pl.ds(row0, ROWS_PER_TILE)])
    plsc.subcore_barrier()
    start(0, 0)
    lax.fori_loop(0, (NCHUNK + 1) // 2, pair_body, 0)
    drain_scatter(0)
    drain_scatter(1)
    plsc.subcore_barrier()
    pltpu.sync_copy(acc.at[pl.ds(row0, ROWS_PER_TILE)], stage_v)
    pltpu.sync_copy(stage_v, out_hbm.at[c, s])


def _sc_scatter(a, b, e, idxi3, idxj3, zeros_tile, hinit):
    mesh = plsc.VectorSubcoreMesh(core_axis_name="c", subcore_axis_name="s")
    kfn = pl.kernel(
        _sc_body,
        out_type=jax.ShapeDtypeStruct((NC, NS, ROWS_PER_TILE, D_SC),
                                      jnp.float32),
        mesh=mesh,
        scratch_types=[
            pltpu.VMEM((NCHUNK, CHUNK), jnp.int32),
            pltpu.VMEM((NCHUNK, CHUNK), jnp.int32),
            pltpu.VMEM((CHUNK, OUT_CHANNELS), jnp.float32),
            pltpu.VMEM((CHUNK, OUT_CHANNELS), jnp.float32),
            pltpu.VMEM((CHUNK, OUT_CHANNELS), jnp.float32),
            pltpu.VMEM((CHUNK, OUT_CHANNELS), jnp.float32),
            pltpu.VMEM((CHUNK, OUT_CHANNELS), jnp.float32),
            pltpu.VMEM((CHUNK, OUT_CHANNELS), jnp.float32),
            pltpu.VMEM((CHUNK, D_SC), jnp.float32),
            pltpu.VMEM((CHUNK, D_SC), jnp.float32),
            pltpu.VMEM((ROWS_PER_TILE, D_SC), jnp.float32),
            pltpu.VMEM_SHARED((N_NODES, D_SC), jnp.float32),
            pltpu.SemaphoreType.DMA,
            pltpu.SemaphoreType.DMA,
            pltpu.SemaphoreType.DMA,
            pltpu.SemaphoreType.DMA,
        ],
        compiler_params=pltpu.CompilerParams(use_tc_tiling_on_sc=False),
    )
    return kfn(a, b, e, idxi3, idxj3, zeros_tile, hinit)


def _fin_body(p_ref, w2e_ref, o_ref):
    t = p_ref[0] + p_ref[1]
    o_ref[...] = lax.dot_general(
        w2e_ref[...], t,
        dimension_numbers=(((0,), (1,)), ((), ())),
        preferred_element_type=jnp.float32,
    )


def _finalize(parts, w2e):
    return pl.pallas_call(
        _fin_body,
        out_shape=jax.ShapeDtypeStruct((OUT_CHANNELS, N_NODES), jnp.float32),
    )(parts, w2e)


def kernel(x, edge_index, edge_attr, W1, b1, W2, b2):
    w1a = W1[:NODE_SIZE]
    w1b = W1[NODE_SIZE:2 * NODE_SIZE]
    w1e = W1[2 * NODE_SIZE:]
    eye4 = jnp.eye(4, dtype=jnp.float32)
    a4, b4, e = _compute_front(
        x.reshape(_N4, 4 * NODE_SIZE), edge_attr.T,
        jnp.kron(eye4, w1a), jnp.kron(eye4, w1b), jnp.kron(eye4, w1e),
        jnp.tile(b1, 4).reshape(1, 128))
    a = a4.reshape(N_NODES, OUT_CHANNELS)
    b = b4.reshape(N_NODES, OUT_CHANNELS)

    idxi3 = edge_index[0].reshape(NW, NCHUNK, CHUNK)
    idxj3 = edge_index[1].reshape(NW, NCHUNK, CHUNK)
    zeros_tile = jnp.zeros((ROWS_PER_TILE, D_SC), jnp.float32)
    hinit = jnp.zeros((CHUNK, D_SC), jnp.float32).at[:, OUT_CHANNELS].set(1.0)

    parts = _sc_scatter(a, b, e, idxi3, idxj3, zeros_tile, hinit)
    parts = parts.reshape(NC, N_NODES, D_SC)

    w2e = jnp.concatenate(
        [W2, b2[None, :], jnp.zeros((D_SC - OUT_CHANNELS - 1, OUT_CHANNELS),
                                    jnp.float32)], axis=0)
    return _finalize(parts, w2e).T


# speedup vs baseline: 16.2080x; 1.0081x over previous
import functools

import jax
import jax.numpy as jnp
from jax import lax
from jax.experimental import pallas as pl
from jax.experimental.pallas import tpu as pltpu
from jax.experimental.pallas import tpu_sc as plsc

N_NODES = 10000
NODE_SIZE = 128
EDGE_SIZE = 16
OUT_CHANNELS = 32
N_EDGES = 320000

NC = 2
NS = 16
NW = NC * NS
E_PER_W = N_EDGES // NW
CHUNK = 125
NCHUNK = E_PER_W // CHUNK
ROWS_PER_TILE = N_NODES // NS
D_SC = 40


_E_ROWS = N_EDGES // 4
_E_BLK = 16000
_N4 = N_NODES // 4


def _front_body(x4_ref, wa4_ref, wb4_ref, eat0_ref, eat1_ref, eat2_ref,
                eat3_ref, we4_ref, b1_ref, a4_ref, b4_ref, e_ref):
    @pl.when(pl.program_id(0) == 0)
    def _():
        xv = x4_ref[...]
        a4_ref[...] = jnp.dot(xv, wa4_ref[...],
                              preferred_element_type=jnp.float32)
        b4_ref[...] = jnp.dot(xv, wb4_ref[...],
                              preferred_element_type=jnp.float32)

    cat = jnp.concatenate(
        [eat0_ref[...], eat1_ref[...], eat2_ref[...], eat3_ref[...]], axis=0)
    e_ref[...] = lax.dot_general(
        cat, we4_ref[...],
        dimension_numbers=(((0,), (0,)), ((), ())),
        preferred_element_type=jnp.float32,
    ) + b1_ref[...]


def _compute_front(x4, edge_attr_t, wa4, wb4, we4, b1row128):
    grid = (_E_ROWS // _E_BLK,)

    def eat_spec(k):
        nblk = _E_ROWS // _E_BLK
        return pl.BlockSpec((EDGE_SIZE, _E_BLK),
                            lambda i, kk=k: (0, nblk * kk + i))

    return pl.pallas_call(
        _front_body,
        grid=grid,
        in_specs=[
            pl.BlockSpec((_N4, 4 * NODE_SIZE), lambda i: (0, 0)),
            pl.BlockSpec((4 * NODE_SIZE, 128), lambda i: (0, 0)),
            pl.BlockSpec((4 * NODE_SIZE, 128), lambda i: (0, 0)),
            eat_spec(0), eat_spec(1), eat_spec(2), eat_spec(3),
            pl.BlockSpec((4 * EDGE_SIZE, 128), lambda i: (0, 0)),
            pl.BlockSpec((1, 128), lambda i: (0, 0)),
        ],
        out_specs=(
            pl.BlockSpec((_N4, 128), lambda i: (0, 0)),
            pl.BlockSpec((_N4, 128), lambda i: (0, 0)),
            pl.BlockSpec((_E_BLK, 128), lambda i: (i, 0)),
        ),
        out_shape=(
            jax.ShapeDtypeStruct((_N4, 128), jnp.float32),
            jax.ShapeDtypeStruct((_N4, 128), jnp.float32),
            jax.ShapeDtypeStruct((_E_ROWS, 128), jnp.float32),
        ),
    )(x4, wa4, wb4, edge_attr_t, edge_attr_t, edge_attr_t, edge_attr_t,
      we4, b1row128)


def _sc_body(a_hbm, b_hbm, e_hbm, idxi_hbm, idxj_hbm, zeros_hbm, hinit_hbm,
             out_hbm, idxi_v, idxj_v, a_buf0, a_buf1, b_buf0, b_buf1,
             e_buf0, e_buf1, h_buf0, h_buf1, stage_v, acc,
             sem_g0, sem_g1, sem_s0, sem_s1):
    c = lax.axis_index("c")
    s = lax.axis_index("s")
    wid = c * NS + s
    a_bufs = (a_buf0, a_buf1)
    b_bufs = (b_buf0, b_buf1)
    e_bufs = (e_buf0, e_buf1)
    h_bufs = (h_buf0, h_buf1)
    sem_g = (sem_g0, sem_g1)
    sem_s = (sem_s0, sem_s1)

    cps = (
        pltpu.async_copy(idxi_hbm.at[wid], idxi_v, sem_g0),
        pltpu.async_copy(idxj_hbm.at[wid], idxj_v, sem_g0),
        pltpu.async_copy(hinit_hbm, h_buf0, sem_g0),
        pltpu.async_copy(hinit_hbm, h_buf1, sem_g0),
        pltpu.async_copy(zeros_hbm, stage_v, sem_g0),
    )
    for cp in cps:
        cp.wait()
    row0 = s * ROWS_PER_TILE
    e_col0 = (wid // 8) * OUT_CHANNELS
    e_row0 = (wid % 8) * E_PER_W

    def start(g, p):
        pltpu.async_copy(a_hbm.at[idxi_v.at[g]], a_bufs[p], sem_g[p])
        pltpu.async_copy(b_hbm.at[idxj_v.at[g]], b_bufs[p], sem_g[p])
        pltpu.async_copy(
            e_hbm.at[pl.ds(e_row0 + g * CHUNK, CHUNK),
                     pl.ds(e_col0, OUT_CHANNELS)],
            e_bufs[p], sem_g[p])

    def drain_gathers(p):
        pltpu.make_async_copy(a_hbm.at[pl.ds(0, CHUNK)], a_bufs[p],
                              sem_g[p]).wait()
        pltpu.make_async_copy(a_hbm.at[pl.ds(0, CHUNK)], b_bufs[p],
                              sem_g[p]).wait()
        pltpu.make_async_copy(
            e_hbm.at[pl.ds(0, CHUNK), pl.ds(0, OUT_CHANNELS)], e_bufs[p],
            sem_g[p]).wait()

    def drain_scatter(p):
        pltpu.make_async_copy(hinit_hbm, h_bufs[p], sem_s[p]).wait()

    def pair_body(gg, carry):
        for p in (0, 1):
            g = 2 * gg + p

            @pl.when(g < NCHUNK)
            def _():
                @pl.when(g + 1 < NCHUNK)
                def _():
                    start(g + 1, 1 - p)

                drain_gathers(p)

                @pl.when(g >= 2)
                def _():
                    drain_scatter(p)

                ab, bb, eb, hb = a_bufs[p], b_bufs[p], e_bufs[p], h_bufs[p]

                @plsc.parallel_loop(0, CHUNK, unroll=5)
                def _(r):
                    for hh in (0, 16):
                        sl = pl.ds(hh, 16)
                        hb[r, sl] = jnp.maximum(
                            ab[r, sl] + bb[r, sl] + eb[r, sl], 0.0)
                pltpu.async_copy(hb, acc.at[idxi_v.at[g]], sem_s[p],
                                 add=True)
        return carry

    pltpu.sync_copy(stage_v, acc.at[pl.ds(row0, ROWS_PER_TILE)])
    plsc.subcore_barrier()
    start(0, 0)
    lax.fori_loop(0, (NCHUNK + 1) // 2, pair_body, 0)
    drain_scatter(0)
    drain_scatter(1)
    plsc.subcore_barrier()
    pltpu.sync_copy(acc.at[pl.ds(row0, ROWS_PER_TILE)], stage_v)
    pltpu.sync_copy(stage_v, out_hbm.at[c, s])


def _sc_scatter(a, b, e, idxi3, idxj3, zeros_tile, hinit):
    mesh = plsc.VectorSubcoreMesh(core_axis_name="c", subcore_axis_name="s")
    kfn = pl.kernel(
        _sc_body,
        out_type=jax.ShapeDtypeStruct((NC, NS, ROWS_PER_TILE, D_SC),
                                      jnp.float32),
        mesh=mesh,
        scratch_types=[
            pltpu.VMEM((NCHUNK, CHUNK), jnp.int32),
            pltpu.VMEM((NCHUNK, CHUNK), jnp.int32),
            pltpu.VMEM((CHUNK, OUT_CHANNELS), jnp.float32),
            pltpu.VMEM((CHUNK, OUT_CHANNELS), jnp.float32),
            pltpu.VMEM((CHUNK, OUT_CHANNELS), jnp.float32),
            pltpu.VMEM((CHUNK, OUT_CHANNELS), jnp.float32),
            pltpu.VMEM((CHUNK, OUT_CHANNELS), jnp.float32),
            pltpu.VMEM((CHUNK, OUT_CHANNELS), jnp.float32),
            pltpu.VMEM((CHUNK, D_SC), jnp.float32),
            pltpu.VMEM((CHUNK, D_SC), jnp.float32),
            pltpu.VMEM((ROWS_PER_TILE, D_SC), jnp.float32),
            pltpu.VMEM_SHARED((N_NODES, D_SC), jnp.float32),
            pltpu.SemaphoreType.DMA,
            pltpu.SemaphoreType.DMA,
            pltpu.SemaphoreType.DMA,
            pltpu.SemaphoreType.DMA,
        ],
        compiler_params=pltpu.CompilerParams(use_tc_tiling_on_sc=False),
    )
    return kfn(a, b, e, idxi3, idxj3, zeros_tile, hinit)


def _fin_body(p_ref, w2e_ref, o_ref):
    t = p_ref[0] + p_ref[1]
    o_ref[...] = lax.dot_general(
        w2e_ref[...], t,
        dimension_numbers=(((0,), (1,)), ((), ())),
        preferred_element_type=jnp.float32,
    )


def _finalize(parts, w2e):
    return pl.pallas_call(
        _fin_body,
        out_shape=jax.ShapeDtypeStruct((OUT_CHANNELS, N_NODES), jnp.float32),
    )(parts, w2e)


def kernel(x, edge_index, edge_attr, W1, b1, W2, b2):
    w1a = W1[:NODE_SIZE]
    w1b = W1[NODE_SIZE:2 * NODE_SIZE]
    w1e = W1[2 * NODE_SIZE:]
    eye4 = jnp.eye(4, dtype=jnp.float32)
    a4, b4, e = _compute_front(
        x.reshape(_N4, 4 * NODE_SIZE), edge_attr.T,
        jnp.kron(eye4, w1a), jnp.kron(eye4, w1b), jnp.kron(eye4, w1e),
        jnp.tile(b1, 4).reshape(1, 128))
    a = a4.reshape(N_NODES, OUT_CHANNELS)
    b = b4.reshape(N_NODES, OUT_CHANNELS)

    idxi3 = edge_index[0].reshape(NW, NCHUNK, CHUNK)
    idxj3 = edge_index[1].reshape(NW, NCHUNK, CHUNK)
    zeros_tile = jnp.zeros((ROWS_PER_TILE, D_SC), jnp.float32)
    hinit = jnp.zeros((CHUNK, D_SC), jnp.float32).at[:, OUT_CHANNELS].set(1.0)

    parts = _sc_scatter(a, b, e, idxi3, idxj3, zeros_tile, hinit)
    parts = parts.reshape(NC, N_NODES, D_SC)

    w2e = jnp.concatenate(
        [W2, b2[None, :], jnp.zeros((D_SC - OUT_CHANNELS - 1, OUT_CHANNELS),
                                    jnp.float32)], axis=0)
    return _finalize(parts, w2e).T
